# baseline (speedup 1.0000x reference)
import jax
import jax.numpy as jnp
from jax.experimental import pallas as pl
from jax.experimental.pallas import tpu as pltpu

N_USER = 88
N_ITEM = 585
N = 673
D = 64
B = 1024
L = 3
NA = 11
NP = NA * D
NROW = 680
G = 4
RB = 176
NEG = -1000000000.0


def _leaky(x, a):
    return jnp.where(x >= 0, x, a * x)


def _mm(a, b):
    return jnp.dot(a, b, preferred_element_type=jnp.float32)


def _mm_t(a, b):
    return jax.lax.dot_general(a, b, (((1,), (1,)), ((), ())),
                               preferred_element_type=jnp.float32)


def _shuffle(t, tpad_ref, seq_ref):
    tpad_ref[0:N, :] = t
    tpad_ref[N:NROW, :] = jnp.zeros((NROW - N, D), jnp.float32)
    bb = jax.lax.broadcasted_iota(jnp.int32, (NA, D), 1)
    for d in range(D):
        q, s = (N * d) // D, (N * d) % D
        w = tpad_ref[q:q + NA + 1, :]
        if s == 0:
            seq_ref[d] = w[0:NA, :]
        else:
            lo = jnp.roll(w[0:NA, :], -s, axis=1)
            hi = jnp.roll(w[1:NA + 1, :], -s, axis=1)
            seq_ref[d] = jnp.where(bb < D - s, lo, hi)


def _mono_kernel(inter_ref, uet_ref, iet_ref, adj_ref,
                 wg_ref, bg_ref, wb_ref, bb_ref,
                 users_ref, pos_ref, neg_ref,
                 c1_ref, c1b_ref, c2_ref, c2b_ref, c3_ref, c3b_ref,
                 pos_out_ref, neg_out_ref,
                 tpad_ref, seq_ref, adj_s, coefs_s, c1_s, c2_s, c3_s):
    g = pl.program_id(0)

    adj_s[pl.ds(g * RB, RB), :] = adj_ref[...]
    c1_s[pl.ds(g * (256 // G), 256 // G), :] = c1_ref[...]
    c2_s[pl.ds(g * (256 // G), 256 // G), :] = c2_ref[...]
    c3_s[pl.ds(g * (256 // G), 256 // G), :] = c3_ref[...]

    ii = jax.lax.broadcasted_iota(jnp.int32, (RB, N), 0) + g * RB
    jj = jax.lax.broadcasted_iota(jnp.int32, (RB, N), 1)
    eye = (ii == jj).astype(jnp.float32)
    mt = inter_ref[...] + eye
    region = (ii < 2 * N_USER) & (jj < 2 * N_USER)
    mt = jnp.where(region & (mt > 0), 1.0, mt)
    e2 = jnp.exp(NEG * (1.0 - mt))
    coefs_blk = e2 / jnp.sum(e2, axis=1, keepdims=True)
    coefs_s[pl.ds(g * RB, RB), 0:N] = coefs_blk
    coefs_s[pl.ds(g * RB, RB), N:NP] = jnp.zeros((RB, NP - N), jnp.float32)

    @pl.when(g == G - 1)
    def _main():
        adj = adj_s[0:N, :]
        coefs_pad = coefs_s[0:N, :]
        ego = jnp.concatenate(
            [jnp.transpose(uet_ref[...]), jnp.transpose(iet_ref[...])],
            axis=0)
        alls = [ego]
        for k in range(L):
            side = _mm(adj, ego)
            sum_emb = _mm(side, wg_ref[k]) + bg_ref[k]
            t = _mm(ego * side, wb_ref[k]) + bb_ref[k]
            _shuffle(t, tpad_ref, seq_ref)
            seq2 = jnp.concatenate(
                [seq_ref[:, a, :] for a in range(NA)], axis=1)
            bi = jnp.transpose(_mm_t(seq2, coefs_pad))
            ego = _leaky(sum_emb + bi, 0.2)
            nrm = jnp.maximum(
                jnp.sqrt(jnp.sum(ego * ego, axis=1, keepdims=True)), 1e-12)
            alls.append(ego / nrm)

        def _lane_row(idx_ref):
            return jnp.concatenate(
                [idx_ref[r:r + 1, :] for r in range(8)], axis=1)

        iu = jax.lax.broadcasted_iota(jnp.int32, (N_USER, B), 0)
        oh_u = (_lane_row(users_ref) == iu).astype(jnp.float32)
        it = jax.lax.broadcasted_iota(jnp.int32, (N_ITEM, B), 0)
        oh_p = (_lane_row(pos_ref) == it).astype(jnp.float32)
        oh_n = (_lane_row(neg_ref) == it).astype(jnp.float32)

        def _gsel(oh, blk):
            return jax.lax.dot_general(oh, blk, (((0,), (0,)), ((), ())),
                                       preferred_element_type=jnp.float32)

        all_cat = jnp.concatenate(alls, axis=1)
        u_g = _gsel(oh_u, all_cat[:N_USER, :])
        p_g = _gsel(oh_p, all_cat[N_USER:, :])
        n_g = _gsel(oh_n, all_cat[N_USER:, :])

        bf16 = jnp.bfloat16
        c1 = c1_s[...].astype(bf16)
        c2 = c2_s[...].astype(bf16)
        c3 = c3_s[...].astype(bf16)

        u_h = c1b_ref[...] + _mm_t(u_g.astype(bf16), c1[:, :256])

        def head(i_g):
            h = jax.nn.relu(u_h + _mm_t(i_g.astype(bf16), c1[:, 256:]))
            h = jax.nn.relu(_mm_t(h.astype(bf16), c2) + c2b_ref[...])
            h = jax.nn.relu(_mm_t(h.astype(bf16), c3) + c3b_ref[...])
            mf = jnp.sum(u_g * i_g, axis=1, keepdims=True)
            return jnp.sum(0.001 * h, axis=1, keepdims=True) + mf

        ieye = jax.lax.broadcasted_iota(jnp.int32, (128, 128), 0)
        jeye = jax.lax.broadcasted_iota(jnp.int32, (128, 128), 1)
        eye128 = (ieye == jeye).astype(jnp.float32)

        def _to_grid(col):
            rows = [jax.lax.dot_general(col[128 * r:128 * (r + 1), :], eye128,
                                        (((0,), (0,)), ((), ())),
                                        preferred_element_type=jnp.float32)
                    for r in range(8)]
            return jnp.concatenate(rows, axis=0)

        pos_out_ref[...] = _to_grid(head(p_g))
        neg_out_ref[...] = _to_grid(head(n_g))


def kernel(users, pos_items, neg_items, norm_adj, interaction, user_emb,
           item_emb, W_gc, b_gc, W_bi, b_bi, conv1_w, conv1_b,
           c1_w, c1_b, c2_w, c2_b, c3_w, c3_b):
    f32 = jnp.float32

    def full(shape):
        return pl.BlockSpec(shape, lambda *_: (0,) * len(shape))

    pos_out, neg_out = pl.pallas_call(
        _mono_kernel,
        grid=(G,),
        out_shape=(jax.ShapeDtypeStruct((8, 128), f32),
                   jax.ShapeDtypeStruct((8, 128), f32)),
        in_specs=[
            pl.BlockSpec((RB, N), lambda i: (i, 0)),
            full((D, N_USER)), full((D, N_ITEM)),
            pl.BlockSpec((RB, N), lambda i: (i, 0)),
            full((L, D, D)), full((L, 1, D)),
            full((L, D, D)), full((L, 1, D)),
            full((8, 128)), full((8, 128)), full((8, 128)),
            pl.BlockSpec((256 // G, 512), lambda i: (i, 0)),
            full((1, 256)),
            pl.BlockSpec((256 // G, 256), lambda i: (i, 0)),
            full((1, 256)),
            pl.BlockSpec((256 // G, 256), lambda i: (i, 0)),
            full((1, 256)),
        ],
        out_specs=(pl.BlockSpec((8, 128), lambda i: (0, 0)),
                   pl.BlockSpec((8, 128), lambda i: (0, 0))),
        scratch_shapes=[
            pltpu.VMEM((NROW, D), f32),
            pltpu.VMEM((D, NA, D), f32),
            pltpu.VMEM((G * RB, N), f32),
            pltpu.VMEM((G * RB, NP), f32),
            pltpu.VMEM((256, 512), f32),
            pltpu.VMEM((256, 256), f32),
            pltpu.VMEM((256, 256), f32),
        ],
    )(
        interaction, user_emb.T, item_emb.T, norm_adj,
        W_gc, b_gc, W_bi, b_bi,
        users.astype(jnp.int32).reshape(8, 128),
        pos_items.astype(jnp.int32).reshape(8, 128),
        neg_items.astype(jnp.int32).reshape(8, 128),
        c1_w, c1_b.reshape(1, 256),
        c2_w, c2_b.reshape(1, 256), c3_w, c3_b.reshape(1, 256))

    return (pos_out.reshape(B), neg_out.reshape(B))


# speedup vs baseline: 1.0687x; 1.0687x over previous
import jax
import jax.numpy as jnp
from jax.experimental import pallas as pl
from jax.experimental.pallas import tpu as pltpu

N_USER = 88
N_ITEM = 585
N = 673
D = 64
B = 1024
L = 3
NA = 11
NROW = 680
NEG = -1000000000.0


def _leaky(x, a):
    return jnp.where(x >= 0, x, a * x)


def _mm(a, b):
    return jnp.dot(a, b, preferred_element_type=jnp.float32)


def _mm_t(a, b):
    return jax.lax.dot_general(a, b, (((1,), (1,)), ((), ())),
                               preferred_element_type=jnp.float32)


def _shuffle(t, tpad_ref, seq_ref):
    tpad_ref[0:N, :] = t
    tpad_ref[N:NROW, :] = jnp.zeros((NROW - N, D), jnp.float32)
    bb = jax.lax.broadcasted_iota(jnp.int32, (NA, D), 1)
    for d in range(D):
        q, s = (N * d) // D, (N * d) % D
        w = tpad_ref[q:q + NA + 1, :]
        if s == 0:
            seq_ref[d] = w[0:NA, :]
        else:
            lo = jnp.roll(w[0:NA, :], -s, axis=1)
            hi = jnp.roll(w[1:NA + 1, :], -s, axis=1)
            seq_ref[d] = jnp.where(bb < D - s, lo, hi)


def _mono_kernel(inter_ref, uet_ref, iet_ref, adj_ref,
                 wg_ref, bg_ref, wb_ref, bb_ref,
                 users_ref, pos_ref, neg_ref,
                 c1_ref, c1b_ref, c2_ref, c2b_ref, c3_ref, c3b_ref,
                 pos_out_ref, neg_out_ref,
                 tpad_ref, seq_ref):
    adj = adj_ref[...]
    ego = jnp.concatenate(
        [jnp.transpose(uet_ref[...]), jnp.transpose(iet_ref[...])], axis=0)

    coefs_pad = None
    alls = [ego]
    for k in range(L):
        side = _mm(adj, ego)
        sum_emb = _mm(side, wg_ref[k]) + bg_ref[k]
        t = _mm(ego * side, wb_ref[k]) + bb_ref[k]
        _shuffle(t, tpad_ref, seq_ref)
        if k == 0:
            ii = jax.lax.broadcasted_iota(jnp.int32, (N, N), 0)
            jj = jax.lax.broadcasted_iota(jnp.int32, (N, N), 1)
            eye = (ii == jj).astype(jnp.float32)
            mt = inter_ref[...] + eye
            region = (ii < 2 * N_USER) & (jj < 2 * N_USER)
            mt = jnp.where(region & (mt > 0), 1.0, mt)
            e2 = jnp.exp(NEG * (1.0 - mt))
            coefs = e2 / jnp.sum(e2, axis=1, keepdims=True)
            coefs_pad = jnp.concatenate(
                [coefs, jnp.zeros((N, NA * D - N), jnp.float32)], axis=1)
        seq2 = jnp.concatenate(
            [seq_ref[:, a, :] for a in range(NA)], axis=1)
        bi = jnp.transpose(_mm_t(seq2, coefs_pad))
        ego = _leaky(sum_emb + bi, 0.2)
        nrm = jnp.maximum(
            jnp.sqrt(jnp.sum(ego * ego, axis=1, keepdims=True)), 1e-12)
        alls.append(ego / nrm)

    def _lane_row(idx_ref):
        return jnp.concatenate(
            [idx_ref[r:r + 1, :] for r in range(8)], axis=1)

    iu = jax.lax.broadcasted_iota(jnp.int32, (N_USER, B), 0)
    oh_u = (_lane_row(users_ref) == iu).astype(jnp.float32)
    it = jax.lax.broadcasted_iota(jnp.int32, (N_ITEM, B), 0)
    oh_p = (_lane_row(pos_ref) == it).astype(jnp.float32)
    oh_n = (_lane_row(neg_ref) == it).astype(jnp.float32)

    def _gsel(oh, blk):
        return jax.lax.dot_general(oh, blk, (((0,), (0,)), ((), ())),
                                   preferred_element_type=jnp.float32)

    all_cat = jnp.concatenate(alls, axis=1)
    u_g = _gsel(oh_u, all_cat[:N_USER, :])
    p_g = _gsel(oh_p, all_cat[N_USER:, :])
    n_g = _gsel(oh_n, all_cat[N_USER:, :])

    bf16 = jnp.bfloat16
    c1 = c1_ref[...].astype(bf16)
    c2 = c2_ref[...].astype(bf16)
    c3 = c3_ref[...].astype(bf16)

    u_h = c1b_ref[...] + _mm_t(u_g.astype(bf16), c1[:, :256])

    def head(i_g):
        h = jax.nn.relu(u_h + _mm_t(i_g.astype(bf16), c1[:, 256:]))
        h = jax.nn.relu(_mm_t(h.astype(bf16), c2) + c2b_ref[...])
        h = jax.nn.relu(_mm_t(h.astype(bf16), c3) + c3b_ref[...])
        mf = jnp.sum(u_g * i_g, axis=1, keepdims=True)
        return jnp.sum(0.001 * h, axis=1, keepdims=True) + mf

    ieye = jax.lax.broadcasted_iota(jnp.int32, (128, 128), 0)
    jeye = jax.lax.broadcasted_iota(jnp.int32, (128, 128), 1)
    eye128 = (ieye == jeye).astype(jnp.float32)

    def _to_grid(col):
        rows = [jax.lax.dot_general(col[128 * r:128 * (r + 1), :], eye128,
                                    (((0,), (0,)), ((), ())),
                                    preferred_element_type=jnp.float32)
                for r in range(8)]
        return jnp.concatenate(rows, axis=0)

    pos_out_ref[...] = _to_grid(head(p_g))
    neg_out_ref[...] = _to_grid(head(n_g))


def kernel(users, pos_items, neg_items, norm_adj, interaction, user_emb,
           item_emb, W_gc, b_gc, W_bi, b_bi, conv1_w, conv1_b,
           c1_w, c1_b, c2_w, c2_b, c3_w, c3_b):
    f32 = jnp.float32

    pos_out, neg_out = pl.pallas_call(
        _mono_kernel,
        out_shape=(jax.ShapeDtypeStruct((8, 128), f32),
                   jax.ShapeDtypeStruct((8, 128), f32)),
        scratch_shapes=[
            pltpu.VMEM((NROW, D), f32),
            pltpu.VMEM((D, NA, D), f32),
        ],
    )(
        interaction, user_emb.T, item_emb.T, norm_adj,
        W_gc, b_gc, W_bi, b_bi,
        users.astype(jnp.int32).reshape(8, 128),
        pos_items.astype(jnp.int32).reshape(8, 128),
        neg_items.astype(jnp.int32).reshape(8, 128),
        c1_w, c1_b.reshape(1, 256),
        c2_w, c2_b.reshape(1, 256), c3_w, c3_b.reshape(1, 256))

    return (pos_out.reshape(B), neg_out.reshape(B))


# speedup vs baseline: 1.0712x; 1.0023x over previous
import jax
import jax.numpy as jnp
from jax.experimental import pallas as pl
from jax.experimental.pallas import tpu as pltpu

N_USER = 88
N_ITEM = 585
N = 673
D = 64
B = 1024
L = 3
NA = 11
NROW = 680
NEG = -1000000000.0


def _leaky(x, a):
    return jnp.where(x >= 0, x, a * x)


def _mm(a, b):
    return jnp.dot(a, b, preferred_element_type=jnp.float32)


def _mm_t(a, b):
    return jax.lax.dot_general(a, b, (((1,), (1,)), ((), ())),
                               preferred_element_type=jnp.float32)


def _shuffle(t, tpad_ref, seq_ref):
    tpad_ref[0:N, :] = t
    tpad_ref[N:NROW, :] = jnp.zeros((NROW - N, D), jnp.float32)
    bb = jax.lax.broadcasted_iota(jnp.int32, (NA, D), 1)
    for d in range(D):
        q, s = (N * d) // D, (N * d) % D
        w = tpad_ref[q:q + NA + 1, :]
        if s == 0:
            seq_ref[d] = w[0:NA, :]
        else:
            lo = jnp.roll(w[0:NA, :], -s, axis=1)
            hi = jnp.roll(w[1:NA + 1, :], -s, axis=1)
            seq_ref[d] = jnp.where(bb < D - s, lo, hi)


def _mono_kernel(inter_ref, uet_ref, iet_ref, adj_ref,
                 wg_ref, bg_ref, wb_ref, bb_ref,
                 users_ref, pos_ref, neg_ref,
                 c1_ref, c1b_ref, c2_ref, c2b_ref, c3_ref, c3b_ref,
                 pos_out_ref, neg_out_ref,
                 tpad_ref, seq_ref):
    adj = adj_ref[...]
    ego = jnp.concatenate(
        [jnp.transpose(uet_ref[...]), jnp.transpose(iet_ref[...])], axis=0)

    coefs_pad = None
    alls = [ego]
    for k in range(L):
        side = _mm(adj, ego)
        sum_emb = _mm(side, wg_ref[k]) + bg_ref[k]
        t = _mm(ego * side, wb_ref[k]) + bb_ref[k]
        _shuffle(t, tpad_ref, seq_ref)
        if k == 0:
            ii = jax.lax.broadcasted_iota(jnp.int32, (N, N), 0)
            jj = jax.lax.broadcasted_iota(jnp.int32, (N, N), 1)
            eye = (ii == jj).astype(jnp.float32)
            mt = inter_ref[...] + eye
            region = (ii < 2 * N_USER) & (jj < 2 * N_USER)
            mt = jnp.where(region & (mt > 0), 1.0, mt)
            e2 = jnp.exp(NEG * (1.0 - mt))
            coefs_pad = e2 / jnp.sum(e2, axis=1, keepdims=True)
        seq2 = jnp.concatenate(
            [seq_ref[:, a, :] for a in range(NA)], axis=1)
        bi = jnp.transpose(_mm_t(seq2[:, 0:N], coefs_pad))
        ego = _leaky(sum_emb + bi, 0.2)
        nrm = jnp.maximum(
            jnp.sqrt(jnp.sum(ego * ego, axis=1, keepdims=True)), 1e-12)
        alls.append(ego / nrm)

    def _lane_row(idx_ref):
        return jnp.concatenate(
            [idx_ref[r:r + 1, :] for r in range(8)], axis=1)

    iu = jax.lax.broadcasted_iota(jnp.int32, (N_USER, B), 0)
    oh_u = (_lane_row(users_ref) == iu).astype(jnp.float32)
    it = jax.lax.broadcasted_iota(jnp.int32, (N_ITEM, B), 0)
    oh_p = (_lane_row(pos_ref) == it).astype(jnp.float32)
    oh_n = (_lane_row(neg_ref) == it).astype(jnp.float32)

    def _gsel(oh, blk):
        return jax.lax.dot_general(oh, blk, (((0,), (0,)), ((), ())),
                                   preferred_element_type=jnp.float32)

    all_cat = jnp.concatenate(alls, axis=1)
    u_g = _gsel(oh_u, all_cat[:N_USER, :])
    p_g = _gsel(oh_p, all_cat[N_USER:, :])
    n_g = _gsel(oh_n, all_cat[N_USER:, :])

    bf16 = jnp.bfloat16
    c1 = c1_ref[...].astype(bf16)
    c2 = c2_ref[...].astype(bf16)
    c3 = c3_ref[...].astype(bf16)

    u_h = c1b_ref[...] + _mm_t(u_g.astype(bf16), c1[:, :256])

    def head(i_g):
        h = jax.nn.relu(u_h + _mm_t(i_g.astype(bf16), c1[:, 256:]))
        h = jax.nn.relu(_mm_t(h.astype(bf16), c2) + c2b_ref[...])
        h = jax.nn.relu(_mm_t(h.astype(bf16), c3) + c3b_ref[...])
        mf = jnp.sum(u_g * i_g, axis=1, keepdims=True)
        return jnp.sum(0.001 * h, axis=1, keepdims=True) + mf

    ieye = jax.lax.broadcasted_iota(jnp.int32, (128, 128), 0)
    jeye = jax.lax.broadcasted_iota(jnp.int32, (128, 128), 1)
    eye128 = (ieye == jeye).astype(jnp.float32)

    def _to_grid(col):
        rows = [jax.lax.dot_general(col[128 * r:128 * (r + 1), :], eye128,
                                    (((0,), (0,)), ((), ())),
                                    preferred_element_type=jnp.float32)
                for r in range(8)]
        return jnp.concatenate(rows, axis=0)

    pos_out_ref[...] = _to_grid(head(p_g))
    neg_out_ref[...] = _to_grid(head(n_g))


def kernel(users, pos_items, neg_items, norm_adj, interaction, user_emb,
           item_emb, W_gc, b_gc, W_bi, b_bi, conv1_w, conv1_b,
           c1_w, c1_b, c2_w, c2_b, c3_w, c3_b):
    f32 = jnp.float32

    pos_out, neg_out = pl.pallas_call(
        _mono_kernel,
        out_shape=(jax.ShapeDtypeStruct((8, 128), f32),
                   jax.ShapeDtypeStruct((8, 128), f32)),
        scratch_shapes=[
            pltpu.VMEM((NROW, D), f32),
            pltpu.VMEM((D, NA, D), f32),
        ],
    )(
        interaction, user_emb.T, item_emb.T, norm_adj,
        W_gc, b_gc, W_bi, b_bi,
        users.astype(jnp.int32).reshape(8, 128),
        pos_items.astype(jnp.int32).reshape(8, 128),
        neg_items.astype(jnp.int32).reshape(8, 128),
        c1_w, c1_b.reshape(1, 256),
        c2_w, c2_b.reshape(1, 256), c3_w, c3_b.reshape(1, 256))

    return (pos_out.reshape(B), neg_out.reshape(B))
